# baseline (speedup 1.0000x reference)
import functools

import jax
import jax.numpy as jnp
from jax import lax
from jax.experimental import pallas as pl
from jax.experimental.pallas import tpu as pltpu
from jax.experimental.pallas import tpu_sc as plsc

_LANES = 16
_NCH = 4


def kernel(node_ids, u, indptr, indices):
    B, S = u.shape
    E = indices.shape[0]
    info = plsc.get_sparse_core_info()
    n_workers = info.num_cores * info.num_subcores
    QW = B // n_workers
    SW = QW * S
    CQ = QW // _NCH
    CS = CQ * S
    NG = CQ // _LANES
    assert B % n_workers == 0 and QW % (_NCH * _LANES) == 0

    mesh = plsc.VectorSubcoreMesh(core_axis_name="c", subcore_axis_name="s")

    @functools.partial(
        pl.kernel,
        mesh=mesh,
        compiler_params=pltpu.CompilerParams(needs_layout_passes=False),
        out_type=[
            jax.ShapeDtypeStruct((B, S), jnp.int32),
            jax.ShapeDtypeStruct((B, S), jnp.int32),
            jax.ShapeDtypeStruct((B, S), jnp.int32),
        ],
        scratch_types=[
            pltpu.VMEM((QW,), jnp.int32),
            pltpu.VMEM((QW,), jnp.int32),
            pltpu.VMEM((QW,), jnp.int32),
            pltpu.VMEM((QW,), jnp.int32),
            pltpu.VMEM((QW,), jnp.float32),
            pltpu.VMEM((QW,), jnp.int32),
            pltpu.VMEM((QW,), jnp.int32),
            pltpu.VMEM((QW,), jnp.int32),
            pltpu.VMEM((QW, S), jnp.float32),
            pltpu.VMEM((SW,), jnp.int32),
            pltpu.VMEM((SW,), jnp.int32),
            pltpu.VMEM((2, CQ, S), jnp.int32),
            pltpu.VMEM((2, CQ, S), jnp.int32),
            pltpu.VMEM((2, CQ, S), jnp.int32),
            pltpu.SemaphoreType.DMA,
            pltpu.SemaphoreType.DMA,
            pltpu.SemaphoreType.DMA,
        ],
    )
    def _run(node_hbm, u_hbm, indptr_hbm, indices_hbm,
             src_hbm, tgt_hbm, msk_hbm,
             ids_v, idsp1_v, start_v, end_v, sdegf_v, sdegm1_v,
             srcval_v, vldq_v, u2_v, gidx_v, tgtf_v,
             src2_v, tgt2_v, msk2_v, gsem, usem, osem):
        wid = lax.axis_index("s") * info.num_cores + lax.axis_index("c")
        qbase = wid * QW

        pltpu.sync_copy(node_hbm.at[pl.ds(qbase, QW)], ids_v)

        iota = lax.iota(jnp.int32, _LANES)

        def fire_u(c):
            sl = pl.ds(c * CQ, CQ)
            return pltpu.async_copy(u_hbm.at[pl.ds(qbase + c * CQ, CQ)],
                                    u2_v.at[sl], usem)

        uh = [None] * _NCH
        uh[0] = fire_u(0)

        for g in range(QW // _LANES):
            sl = pl.ds(g * _LANES, _LANES)
            idsp1_v[sl] = ids_v[sl] + 1

        h1 = pltpu.async_copy(indptr_hbm.at[ids_v], start_v, gsem)
        h2 = pltpu.async_copy(indptr_hbm.at[idsp1_v], end_v, gsem)
        h1.wait()
        h2.wait()

        for g in range(QW // _LANES):
            sl = pl.ds(g * _LANES, _LANES)
            deg = end_v[sl] - start_v[sl]
            sdeg = jnp.maximum(deg, 1)
            sdegf_v[sl] = sdeg.astype(jnp.float32)
            sdegm1_v[sl] = sdeg - 1
            valid = deg > 0
            srcval_v[sl] = jnp.where(valid, ids_v[sl], -1)
            vldq_v[sl] = valid.astype(jnp.int32)

        def ph1(c):
            buf = c % 2
            for g in range(NG):
                qloc = g * _LANES + iota
                qvw = c * CQ + qloc
                st = plsc.load_gather(start_v, [qvw])
                sdf = plsc.load_gather(sdegf_v, [qvw])
                sdm1 = plsc.load_gather(sdegm1_v, [qvw])
                sval = plsc.load_gather(srcval_v, [qvw])
                vq = plsc.load_gather(vldq_v, [qvw])
                tb = qvw * S
                for s in range(S):
                    sfull = jnp.full((_LANES,), s, jnp.int32)
                    uv = plsc.load_gather(u2_v, [qvw, sfull])
                    off = (uv * sdf).astype(jnp.int32)
                    off = jnp.minimum(off, sdm1)
                    gi = jnp.minimum(st + off, E - 1)
                    plsc.store_scatter(gidx_v, [tb + s], gi)
                    plsc.store_scatter(src2_v.at[buf], [qloc, sfull], sval)
                    plsc.store_scatter(msk2_v.at[buf], [qloc, sfull], vq)

        def ph2(c):
            buf = c % 2
            for g in range(NG):
                qloc = g * _LANES + iota
                qvw = c * CQ + qloc
                vq = plsc.load_gather(vldq_v, [qvw])
                vmask = vq > 0
                tb = qvw * S
                for s in range(S):
                    sfull = jnp.full((_LANES,), s, jnp.int32)
                    tv = plsc.load_gather(tgtf_v, [tb + s])
                    tv = jnp.where(vmask, tv, -1)
                    plsc.store_scatter(tgt2_v.at[buf], [qloc, sfull], tv)

        def fire_gather(c):
            sl = pl.ds(c * CS, CS)
            return pltpu.async_copy(indices_hbm.at[gidx_v.at[sl]],
                                    tgtf_v.at[sl], gsem)

        def fire_out(c):
            buf = c % 2
            rsl = pl.ds(qbase + c * CQ, CQ)
            return [
                pltpu.async_copy(src2_v.at[buf], src_hbm.at[rsl], osem),
                pltpu.async_copy(tgt2_v.at[buf], tgt_hbm.at[rsl], osem),
                pltpu.async_copy(msk2_v.at[buf], msk_hbm.at[rsl], osem),
            ]

        gh = [None] * _NCH
        oh = [None] * _NCH
        for c in range(_NCH):
            if c >= 2:
                for h in oh[c - 2]:
                    h.wait()
            uh[c].wait()
            if c + 1 < _NCH:
                uh[c + 1] = fire_u(c + 1)
            ph1(c)
            gh[c] = fire_gather(c)
            if c >= 1:
                gh[c - 1].wait()
                ph2(c - 1)
                oh[c - 1] = fire_out(c - 1)
        gh[_NCH - 1].wait()
        ph2(_NCH - 1)
        oh[_NCH - 1] = fire_out(_NCH - 1)
        for c in (_NCH - 2, _NCH - 1):
            for h in oh[c]:
                h.wait()

    src, tgt, msk = _run(node_ids, u, indptr, indices)
    return (src, tgt, msk.astype(bool))


# speedup vs baseline: 1.0206x; 1.0206x over previous
import functools

import jax
import jax.numpy as jnp
from jax import lax
from jax.experimental import pallas as pl
from jax.experimental.pallas import tpu as pltpu
from jax.experimental.pallas import tpu_sc as plsc

_LANES = 16
_NCH = 4


def kernel(node_ids, u, indptr, indices):
    B, S = u.shape
    E = indices.shape[0]
    info = plsc.get_sparse_core_info()
    n_workers = info.num_cores * info.num_subcores
    QW = B // n_workers
    SW = QW * S
    CQ = QW // _NCH
    CS = CQ * S
    NG = CQ // _LANES
    assert B % n_workers == 0 and QW % (_NCH * _LANES) == 0

    mesh = plsc.VectorSubcoreMesh(core_axis_name="c", subcore_axis_name="s")

    @functools.partial(
        pl.kernel,
        mesh=mesh,
        compiler_params=pltpu.CompilerParams(needs_layout_passes=False),
        out_type=[
            jax.ShapeDtypeStruct((B, S), jnp.int32),
            jax.ShapeDtypeStruct((B, S), jnp.int32),
            jax.ShapeDtypeStruct((B, S), jnp.int32),
        ],
        scratch_types=[
            pltpu.VMEM((QW,), jnp.int32),
            pltpu.VMEM((QW,), jnp.int32),
            pltpu.VMEM((QW,), jnp.int32),
            pltpu.VMEM((QW,), jnp.int32),
            pltpu.VMEM((QW,), jnp.float32),
            pltpu.VMEM((QW,), jnp.int32),
            pltpu.VMEM((QW,), jnp.int32),
            pltpu.VMEM((QW,), jnp.int32),
            pltpu.VMEM((QW, S), jnp.float32),
            pltpu.VMEM((SW,), jnp.int32),
            pltpu.VMEM((SW,), jnp.int32),
            pltpu.VMEM((2, CQ, S), jnp.int32),
            pltpu.VMEM((2, CQ, S), jnp.int32),
            pltpu.VMEM((2, CQ, S), jnp.int32),
            pltpu.SemaphoreType.DMA,
            pltpu.SemaphoreType.DMA,
            pltpu.SemaphoreType.DMA,
        ],
    )
    def _run(node_hbm, u_hbm, indptr_hbm, indices_hbm,
             src_hbm, tgt_hbm, msk_hbm,
             ids_v, idsp1_v, start_v, end_v, sdegf_v, sdegm1_v,
             srcval_v, vldq_v, u2_v, gidx_v, tgtf_v,
             src2_v, tgt2_v, msk2_v, gsem, usem, osem):
        wid = lax.axis_index("s") * info.num_cores + lax.axis_index("c")
        qbase = wid * QW

        pltpu.sync_copy(node_hbm.at[pl.ds(qbase, QW)], ids_v)

        iota = lax.iota(jnp.int32, _LANES)

        def fire_u(c):
            sl = pl.ds(c * CQ, CQ)
            return pltpu.async_copy(u_hbm.at[pl.ds(qbase + c * CQ, CQ)],
                                    u2_v.at[sl], usem)

        uh = [None] * _NCH
        uh[0] = fire_u(0)

        def mk_idsp1(g, carry):
            sl = pl.ds(g * _LANES, _LANES)
            idsp1_v[sl] = ids_v[sl] + 1
            return carry

        lax.fori_loop(0, QW // _LANES, mk_idsp1, 0, unroll=2)

        h1 = pltpu.async_copy(indptr_hbm.at[ids_v], start_v, gsem)
        h2 = pltpu.async_copy(indptr_hbm.at[idsp1_v], end_v, gsem)
        h1.wait()
        h2.wait()

        def precompute(g, carry):
            sl = pl.ds(g * _LANES, _LANES)
            deg = end_v[sl] - start_v[sl]
            sdeg = jnp.maximum(deg, 1)
            sdegf_v[sl] = sdeg.astype(jnp.float32)
            sdegm1_v[sl] = sdeg - 1
            valid = deg > 0
            srcval_v[sl] = jnp.where(valid, ids_v[sl], -1)
            vldq_v[sl] = valid.astype(jnp.int32)
            return carry

        lax.fori_loop(0, QW // _LANES, precompute, 0, unroll=2)

        def ph1(c):
            buf = c % 2

            def group(g, carry):
                qloc = g * _LANES + iota
                qvw = c * CQ + qloc
                st = plsc.load_gather(start_v, [qvw])
                sdf = plsc.load_gather(sdegf_v, [qvw])
                sdm1 = plsc.load_gather(sdegm1_v, [qvw])
                sval = plsc.load_gather(srcval_v, [qvw])
                vq = plsc.load_gather(vldq_v, [qvw])
                tb = qvw * S
                for s in range(S):
                    sfull = jnp.full((_LANES,), s, jnp.int32)
                    uv = plsc.load_gather(u2_v, [qvw, sfull])
                    off = (uv * sdf).astype(jnp.int32)
                    off = jnp.minimum(off, sdm1)
                    gi = jnp.minimum(st + off, E - 1)
                    plsc.store_scatter(gidx_v, [tb + s], gi)
                    plsc.store_scatter(src2_v.at[buf], [qloc, sfull], sval)
                    plsc.store_scatter(msk2_v.at[buf], [qloc, sfull], vq)
                return carry

            lax.fori_loop(0, NG, group, 0)

        def ph2(c):
            buf = c % 2

            def group(g, carry):
                qloc = g * _LANES + iota
                qvw = c * CQ + qloc
                vq = plsc.load_gather(vldq_v, [qvw])
                vmask = vq > 0
                tb = qvw * S
                for s in range(S):
                    sfull = jnp.full((_LANES,), s, jnp.int32)
                    tv = plsc.load_gather(tgtf_v, [tb + s])
                    tv = jnp.where(vmask, tv, -1)
                    plsc.store_scatter(tgt2_v.at[buf], [qloc, sfull], tv)
                return carry

            lax.fori_loop(0, NG, group, 0)

        def fire_gather(c):
            sl = pl.ds(c * CS, CS)
            return pltpu.async_copy(indices_hbm.at[gidx_v.at[sl]],
                                    tgtf_v.at[sl], gsem)

        def fire_out(c):
            buf = c % 2
            rsl = pl.ds(qbase + c * CQ, CQ)
            return [
                pltpu.async_copy(src2_v.at[buf], src_hbm.at[rsl], osem),
                pltpu.async_copy(tgt2_v.at[buf], tgt_hbm.at[rsl], osem),
                pltpu.async_copy(msk2_v.at[buf], msk_hbm.at[rsl], osem),
            ]

        gh = [None] * _NCH
        oh = [None] * _NCH
        for c in range(_NCH):
            if c >= 2:
                for h in oh[c - 2]:
                    h.wait()
            uh[c].wait()
            if c + 1 < _NCH:
                uh[c + 1] = fire_u(c + 1)
            ph1(c)
            gh[c] = fire_gather(c)
            if c >= 1:
                gh[c - 1].wait()
                ph2(c - 1)
                oh[c - 1] = fire_out(c - 1)
        gh[_NCH - 1].wait()
        ph2(_NCH - 1)
        oh[_NCH - 1] = fire_out(_NCH - 1)
        for c in (_NCH - 2, _NCH - 1):
            for h in oh[c]:
                h.wait()

    src, tgt, msk = _run(node_ids, u, indptr, indices)
    return (src, tgt, msk.astype(bool))


# speedup vs baseline: 1.1340x; 1.1112x over previous
import functools

import jax
import jax.numpy as jnp
from jax import lax
from jax.experimental import pallas as pl
from jax.experimental.pallas import tpu as pltpu
from jax.experimental.pallas import tpu_sc as plsc

_LANES = 16
_NCH = 4


def kernel(node_ids, u, indptr, indices):
    B, S = u.shape
    E = indices.shape[0]
    info = plsc.get_sparse_core_info()
    n_workers = info.num_cores * info.num_subcores
    QW = B // n_workers
    SW = QW * S
    CQ = QW // _NCH
    CS = CQ * S
    NG = CQ // _LANES
    assert B % n_workers == 0 and QW % (_NCH * _LANES) == 0

    mesh = plsc.VectorSubcoreMesh(core_axis_name="c", subcore_axis_name="s")

    @functools.partial(
        pl.kernel,
        mesh=mesh,
        compiler_params=pltpu.CompilerParams(needs_layout_passes=False),
        out_type=[
            jax.ShapeDtypeStruct((B, S), jnp.int32),
            jax.ShapeDtypeStruct((B, S), jnp.int32),
            jax.ShapeDtypeStruct((B, S), jnp.int32),
        ],
        scratch_types=[
            pltpu.VMEM((QW,), jnp.int32),
            pltpu.VMEM((QW,), jnp.int32),
            pltpu.VMEM((QW,), jnp.int32),
            pltpu.VMEM((QW,), jnp.int32),
            pltpu.VMEM((QW,), jnp.float32),
            pltpu.VMEM((QW,), jnp.int32),
            pltpu.VMEM((QW,), jnp.int32),
            pltpu.VMEM((QW,), jnp.int32),
            pltpu.VMEM((QW, S), jnp.float32),
            pltpu.VMEM((SW,), jnp.int32),
            pltpu.VMEM((SW,), jnp.int32),
            pltpu.VMEM((2, CQ, S), jnp.int32),
            pltpu.VMEM((2, CQ, S), jnp.int32),
            pltpu.VMEM((2, CQ, S), jnp.int32),
            pltpu.SemaphoreType.DMA,
            pltpu.SemaphoreType.DMA,
            pltpu.SemaphoreType.DMA,
        ],
    )
    def _run(node_hbm, u_hbm, indptr_hbm, indices_hbm,
             src_hbm, tgt_hbm, msk_hbm,
             ids_v, idsp1_v, start_v, end_v, sdegf_v, sdegm1_v,
             srcval_v, vldq_v, u2_v, gidx_v, tgtf_v,
             src2_v, tgt2_v, msk2_v, gsem, usem, osem):
        wid = lax.axis_index("s") * info.num_cores + lax.axis_index("c")
        qbase = wid * QW

        pltpu.sync_copy(node_hbm.at[pl.ds(qbase, QW)], ids_v)

        iota = lax.iota(jnp.int32, _LANES)

        def fire_u(c):
            sl = pl.ds(c * CQ, CQ)
            return pltpu.async_copy(u_hbm.at[pl.ds(qbase + c * CQ, CQ)],
                                    u2_v.at[sl], usem)

        uh = [None] * _NCH
        uh[0] = fire_u(0)

        def mk_idsp1(g, carry):
            sl = pl.ds(g * _LANES, _LANES)
            idsp1_v[sl] = ids_v[sl] + 1
            return carry

        lax.fori_loop(0, QW // _LANES, mk_idsp1, 0, unroll=2)

        h1 = pltpu.async_copy(indptr_hbm.at[ids_v], start_v, gsem)
        h2 = pltpu.async_copy(indptr_hbm.at[idsp1_v], end_v, gsem)
        h1.wait()
        h2.wait()

        def precompute(g, carry):
            sl = pl.ds(g * _LANES, _LANES)
            deg = end_v[sl] - start_v[sl]
            sdeg = jnp.maximum(deg, 1)
            sdegf_v[sl] = sdeg.astype(jnp.float32)
            sdegm1_v[sl] = sdeg - 1
            valid = deg > 0
            srcval_v[sl] = jnp.where(valid, ids_v[sl], -1)
            vldq_v[sl] = valid.astype(jnp.int32)
            return carry

        lax.fori_loop(0, QW // _LANES, precompute, 0, unroll=2)

        def ph1(c):
            buf = c % 2
            s0 = c * CS

            def body(i, carry):
                t0 = s0 + i * _LANES
                tvec = t0 + iota
                qv = lax.div(tvec, S)
                sv = tvec - qv * S
                qloc = qv - (c * CQ)
                st = plsc.load_gather(start_v, [qv])
                sdf = plsc.load_gather(sdegf_v, [qv])
                sdm1 = plsc.load_gather(sdegm1_v, [qv])
                sval = plsc.load_gather(srcval_v, [qv])
                vq = plsc.load_gather(vldq_v, [qv])
                uv = plsc.load_gather(u2_v, [qv, sv])
                off = (uv * sdf).astype(jnp.int32)
                off = jnp.minimum(off, sdm1)
                gidx_v[pl.ds(t0, _LANES)] = jnp.minimum(st + off, E - 1)
                plsc.store_scatter(src2_v.at[buf], [qloc, sv], sval)
                plsc.store_scatter(msk2_v.at[buf], [qloc, sv], vq)
                return carry

            lax.fori_loop(0, CS // _LANES, body, 0, unroll=4)

        def ph2(c):
            buf = c % 2
            s0 = c * CS

            def body(i, carry):
                t0 = s0 + i * _LANES
                tsl = pl.ds(t0, _LANES)
                tvec = t0 + iota
                qv = lax.div(tvec, S)
                sv = tvec - qv * S
                qloc = qv - (c * CQ)
                vq = plsc.load_gather(vldq_v, [qv])
                tv = jnp.where(vq > 0, tgtf_v[tsl], -1)
                plsc.store_scatter(tgt2_v.at[buf], [qloc, sv], tv)
                return carry

            lax.fori_loop(0, CS // _LANES, body, 0, unroll=4)

        def fire_gather(c):
            sl = pl.ds(c * CS, CS)
            return pltpu.async_copy(indices_hbm.at[gidx_v.at[sl]],
                                    tgtf_v.at[sl], gsem)

        def fire_out(c):
            buf = c % 2
            rsl = pl.ds(qbase + c * CQ, CQ)
            return [
                pltpu.async_copy(src2_v.at[buf], src_hbm.at[rsl], osem),
                pltpu.async_copy(tgt2_v.at[buf], tgt_hbm.at[rsl], osem),
                pltpu.async_copy(msk2_v.at[buf], msk_hbm.at[rsl], osem),
            ]

        gh = [None] * _NCH
        oh = [None] * _NCH
        for c in range(_NCH):
            if c >= 2:
                for h in oh[c - 2]:
                    h.wait()
            uh[c].wait()
            if c + 1 < _NCH:
                uh[c + 1] = fire_u(c + 1)
            ph1(c)
            gh[c] = fire_gather(c)
            if c >= 1:
                gh[c - 1].wait()
                ph2(c - 1)
                oh[c - 1] = fire_out(c - 1)
        gh[_NCH - 1].wait()
        ph2(_NCH - 1)
        oh[_NCH - 1] = fire_out(_NCH - 1)
        for c in (_NCH - 2, _NCH - 1):
            for h in oh[c]:
                h.wait()

    src, tgt, msk = _run(node_ids, u, indptr, indices)
    return (src, tgt, msk.astype(bool))


# speedup vs baseline: 1.1577x; 1.0209x over previous
import functools

import jax
import jax.numpy as jnp
from jax import lax
from jax.experimental import pallas as pl
from jax.experimental.pallas import tpu as pltpu
from jax.experimental.pallas import tpu_sc as plsc

_LANES = 16
_NCH = 4


def kernel(node_ids, u, indptr, indices):
    B, S = u.shape
    E = indices.shape[0]
    info = plsc.get_sparse_core_info()
    n_workers = info.num_cores * info.num_subcores
    QW = B // n_workers
    SW = QW * S
    CQ = QW // _NCH
    CS = CQ * S
    NG = CQ // _LANES
    assert B % n_workers == 0 and QW % (_NCH * _LANES) == 0

    mesh = plsc.VectorSubcoreMesh(core_axis_name="c", subcore_axis_name="s")

    @functools.partial(
        pl.kernel,
        mesh=mesh,
        compiler_params=pltpu.CompilerParams(needs_layout_passes=False),
        out_type=[
            jax.ShapeDtypeStruct((B, S), jnp.int32),
            jax.ShapeDtypeStruct((B, S), jnp.int32),
            jax.ShapeDtypeStruct((B, S), jnp.int32),
        ],
        scratch_types=[
            pltpu.VMEM((QW,), jnp.int32),
            pltpu.VMEM((QW,), jnp.int32),
            pltpu.VMEM((QW,), jnp.int32),
            pltpu.VMEM((QW,), jnp.int32),
            pltpu.VMEM((QW,), jnp.float32),
            pltpu.VMEM((QW,), jnp.int32),
            pltpu.VMEM((QW,), jnp.int32),
            pltpu.VMEM((QW,), jnp.int32),
            pltpu.VMEM((QW, S), jnp.float32),
            pltpu.VMEM((SW,), jnp.int32),
            pltpu.VMEM((SW,), jnp.int32),
            pltpu.VMEM((2, CQ, S), jnp.int32),
            pltpu.VMEM((2, CQ, S), jnp.int32),
            pltpu.VMEM((2, CQ, S), jnp.int32),
            pltpu.SemaphoreType.DMA,
            pltpu.SemaphoreType.DMA,
            pltpu.SemaphoreType.DMA,
        ],
    )
    def _run(node_hbm, u_hbm, indptr_hbm, indices_hbm,
             src_hbm, tgt_hbm, msk_hbm,
             ids_v, idsp1_v, start_v, end_v, sdegf_v, sdegm1_v,
             srcval_v, vldq_v, u2_v, gidx_v, tgtf_v,
             src2_v, tgt2_v, msk2_v, gsem, usem, osem):
        wid = lax.axis_index("s") * info.num_cores + lax.axis_index("c")
        qbase = wid * QW

        pltpu.sync_copy(node_hbm.at[pl.ds(qbase, QW)], ids_v)

        iota = lax.iota(jnp.int32, _LANES)

        def fire_u(c):
            sl = pl.ds(c * CQ, CQ)
            return pltpu.async_copy(u_hbm.at[pl.ds(qbase + c * CQ, CQ)],
                                    u2_v.at[sl], usem)

        uh = [None] * _NCH
        uh[0] = fire_u(0)

        def mk_idsp1(g, carry):
            sl = pl.ds(g * _LANES, _LANES)
            idsp1_v[sl] = ids_v[sl] + 1
            return carry

        lax.fori_loop(0, QW // _LANES, mk_idsp1, 0, unroll=2)

        h1 = pltpu.async_copy(indptr_hbm.at[ids_v], start_v, gsem)
        h2 = pltpu.async_copy(indptr_hbm.at[idsp1_v], end_v, gsem)
        h1.wait()
        h2.wait()

        def precompute(g, carry):
            sl = pl.ds(g * _LANES, _LANES)
            deg = end_v[sl] - start_v[sl]
            sdeg = jnp.maximum(deg, 1)
            sdegf_v[sl] = sdeg.astype(jnp.float32)
            sdegm1_v[sl] = sdeg - 1
            valid = deg > 0
            srcval_v[sl] = jnp.where(valid, ids_v[sl], -1)
            vldq_v[sl] = valid.astype(jnp.int32)
            return carry

        lax.fori_loop(0, QW // _LANES, precompute, 0, unroll=2)

        def ph1(c):
            buf = c % 2
            s0 = c * CS

            def body(i, carry):
                t0 = s0 + i * _LANES
                tvec = t0 + iota
                qv = lax.div(tvec, S)
                sv = tvec - qv * S
                qloc = qv - (c * CQ)
                st = plsc.load_gather(start_v, [qv])
                sdf = plsc.load_gather(sdegf_v, [qv])
                sdm1 = plsc.load_gather(sdegm1_v, [qv])
                sval = plsc.load_gather(srcval_v, [qv])
                vq = plsc.load_gather(vldq_v, [qv])
                uv = plsc.load_gather(u2_v, [qv, sv])
                off = (uv * sdf).astype(jnp.int32)
                off = jnp.minimum(off, sdm1)
                gidx_v[pl.ds(t0, _LANES)] = jnp.minimum(st + off, E - 1)
                plsc.store_scatter(src2_v.at[buf], [qloc, sv], sval)
                plsc.store_scatter(msk2_v.at[buf], [qloc, sv], vq)
                return carry

            lax.fori_loop(0, CS // _LANES, body, 0, unroll=2)

        def ph2(c):
            buf = c % 2
            s0 = c * CS

            def body(i, carry):
                t0 = s0 + i * _LANES
                tsl = pl.ds(t0, _LANES)
                tvec = t0 + iota
                qv = lax.div(tvec, S)
                sv = tvec - qv * S
                qloc = qv - (c * CQ)
                vq = plsc.load_gather(vldq_v, [qv])
                tv = jnp.where(vq > 0, tgtf_v[tsl], -1)
                plsc.store_scatter(tgt2_v.at[buf], [qloc, sv], tv)
                return carry

            lax.fori_loop(0, CS // _LANES, body, 0, unroll=2)

        def fire_gather(c):
            sl = pl.ds(c * CS, CS)
            return pltpu.async_copy(indices_hbm.at[gidx_v.at[sl]],
                                    tgtf_v.at[sl], gsem)

        def fire_out(c):
            buf = c % 2
            rsl = pl.ds(qbase + c * CQ, CQ)
            return [
                pltpu.async_copy(src2_v.at[buf], src_hbm.at[rsl], osem),
                pltpu.async_copy(tgt2_v.at[buf], tgt_hbm.at[rsl], osem),
                pltpu.async_copy(msk2_v.at[buf], msk_hbm.at[rsl], osem),
            ]

        gh = [None] * _NCH
        oh = [None] * _NCH
        for c in range(_NCH):
            if c >= 2:
                for h in oh[c - 2]:
                    h.wait()
            uh[c].wait()
            if c + 1 < _NCH:
                uh[c + 1] = fire_u(c + 1)
            ph1(c)
            gh[c] = fire_gather(c)
            if c >= 1:
                gh[c - 1].wait()
                ph2(c - 1)
                oh[c - 1] = fire_out(c - 1)
        gh[_NCH - 1].wait()
        ph2(_NCH - 1)
        oh[_NCH - 1] = fire_out(_NCH - 1)
        for c in (_NCH - 2, _NCH - 1):
            for h in oh[c]:
                h.wait()

    src, tgt, msk = _run(node_ids, u, indptr, indices)
    return (src, tgt, msk.astype(bool))


# speedup vs baseline: 1.1927x; 1.0302x over previous
import functools

import jax
import jax.numpy as jnp
from jax import lax
from jax.experimental import pallas as pl
from jax.experimental.pallas import tpu as pltpu
from jax.experimental.pallas import tpu_sc as plsc

_LANES = 16
_NCH = 4


def kernel(node_ids, u, indptr, indices):
    B, S = u.shape
    E = indices.shape[0]
    info = plsc.get_sparse_core_info()
    n_workers = info.num_cores * info.num_subcores
    QW = B // n_workers
    SW = QW * S
    CQ = QW // _NCH
    CS = CQ * S
    NG = CQ // _LANES
    assert B % n_workers == 0 and QW % (_NCH * _LANES) == 0

    mesh = plsc.VectorSubcoreMesh(core_axis_name="c", subcore_axis_name="s")

    @functools.partial(
        pl.kernel,
        mesh=mesh,
        compiler_params=pltpu.CompilerParams(needs_layout_passes=False),
        out_type=[
            jax.ShapeDtypeStruct((B, S), jnp.int32),
            jax.ShapeDtypeStruct((B, S), jnp.int32),
            jax.ShapeDtypeStruct((B, S), jnp.int32),
        ],
        scratch_types=[
            pltpu.VMEM((QW,), jnp.int32),
            pltpu.VMEM((QW,), jnp.int32),
            pltpu.VMEM((QW,), jnp.int32),
            pltpu.VMEM((QW,), jnp.int32),
            pltpu.VMEM((QW,), jnp.float32),
            pltpu.VMEM((QW,), jnp.int32),
            pltpu.VMEM((QW,), jnp.int32),
            pltpu.VMEM((QW,), jnp.int32),
            pltpu.VMEM((QW, S), jnp.float32),
            pltpu.VMEM((SW,), jnp.int32),
            pltpu.VMEM((SW,), jnp.int32),
            pltpu.VMEM((2, CQ, S), jnp.int32),
            pltpu.VMEM((2, CQ, S), jnp.int32),
            pltpu.VMEM((2, CQ, S), jnp.int32),
            pltpu.SemaphoreType.DMA,
            pltpu.SemaphoreType.DMA,
            pltpu.SemaphoreType.DMA,
            pltpu.SemaphoreType.DMA,
            pltpu.SemaphoreType.DMA,
        ],
    )
    def _run(node_hbm, u_hbm, indptr_hbm, indices_hbm,
             src_hbm, tgt_hbm, msk_hbm,
             ids_v, idsp1_v, start_v, end_v, sdegf_v, sdegm1_v,
             srcval_v, vldq_v, u2_v, gidx_v, tgtf_v,
             src2_v, tgt2_v, msk2_v, gsemA, gsemB, usem, osemA, osemB):
        gsem2 = (gsemA, gsemB)
        osem2 = (osemA, osemB)
        wid = lax.axis_index("s") * info.num_cores + lax.axis_index("c")
        qbase = wid * QW

        pltpu.sync_copy(node_hbm.at[pl.ds(qbase, QW)], ids_v)

        iota = lax.iota(jnp.int32, _LANES)

        def fire_u(c):
            sl = pl.ds(c * CQ, CQ)
            return pltpu.async_copy(u_hbm.at[pl.ds(qbase + c * CQ, CQ)],
                                    u2_v.at[sl], usem)

        uh = [None] * _NCH
        uh[0] = fire_u(0)

        def mk_idsp1(g, carry):
            sl = pl.ds(g * _LANES, _LANES)
            idsp1_v[sl] = ids_v[sl] + 1
            return carry

        lax.fori_loop(0, QW // _LANES, mk_idsp1, 0, unroll=2)

        h1 = pltpu.async_copy(indptr_hbm.at[ids_v], start_v, gsemA)
        h2 = pltpu.async_copy(indptr_hbm.at[idsp1_v], end_v, gsemB)
        h1.wait()
        h2.wait()

        def precompute(g, carry):
            sl = pl.ds(g * _LANES, _LANES)
            deg = end_v[sl] - start_v[sl]
            sdeg = jnp.maximum(deg, 1)
            sdegf_v[sl] = sdeg.astype(jnp.float32)
            sdegm1_v[sl] = sdeg - 1
            valid = deg > 0
            srcval_v[sl] = jnp.where(valid, ids_v[sl], -1)
            vldq_v[sl] = valid.astype(jnp.int32)
            return carry

        lax.fori_loop(0, QW // _LANES, precompute, 0, unroll=2)

        def ph1(c):
            buf = c % 2
            s0 = c * CS

            def body(i, carry):
                t0 = s0 + i * _LANES
                tvec = t0 + iota
                qv = lax.div(tvec, S)
                sv = tvec - qv * S
                qloc = qv - (c * CQ)
                st = plsc.load_gather(start_v, [qv])
                sdf = plsc.load_gather(sdegf_v, [qv])
                sdm1 = plsc.load_gather(sdegm1_v, [qv])
                sval = plsc.load_gather(srcval_v, [qv])
                vq = plsc.load_gather(vldq_v, [qv])
                uv = plsc.load_gather(u2_v, [qv, sv])
                off = (uv * sdf).astype(jnp.int32)
                off = jnp.minimum(off, sdm1)
                gidx_v[pl.ds(t0, _LANES)] = jnp.minimum(st + off, E - 1)
                plsc.store_scatter(src2_v.at[buf], [qloc, sv], sval)
                plsc.store_scatter(msk2_v.at[buf], [qloc, sv], vq)
                return carry

            lax.fori_loop(0, CS // _LANES, body, 0, unroll=2)

        def ph2(c):
            buf = c % 2
            s0 = c * CS

            def body(i, carry):
                t0 = s0 + i * _LANES
                tsl = pl.ds(t0, _LANES)
                tvec = t0 + iota
                qv = lax.div(tvec, S)
                sv = tvec - qv * S
                qloc = qv - (c * CQ)
                vq = plsc.load_gather(vldq_v, [qv])
                tv = jnp.where(vq > 0, tgtf_v[tsl], -1)
                plsc.store_scatter(tgt2_v.at[buf], [qloc, sv], tv)
                return carry

            lax.fori_loop(0, CS // _LANES, body, 0, unroll=2)

        def fire_gather(c):
            sl = pl.ds(c * CS, CS)
            return pltpu.async_copy(indices_hbm.at[gidx_v.at[sl]],
                                    tgtf_v.at[sl], gsem2[c % 2])

        def fire_out(c):
            buf = c % 2
            rsl = pl.ds(qbase + c * CQ, CQ)
            return [
                pltpu.async_copy(src2_v.at[buf], src_hbm.at[rsl], osem2[buf]),
                pltpu.async_copy(tgt2_v.at[buf], tgt_hbm.at[rsl], osem2[buf]),
                pltpu.async_copy(msk2_v.at[buf], msk_hbm.at[rsl], osem2[buf]),
            ]

        gh = [None] * _NCH
        oh = [None] * _NCH
        for c in range(_NCH):
            if c >= 2:
                for h in oh[c - 2]:
                    h.wait()
            uh[c].wait()
            if c + 1 < _NCH:
                uh[c + 1] = fire_u(c + 1)
            ph1(c)
            gh[c] = fire_gather(c)
            if c >= 1:
                gh[c - 1].wait()
                ph2(c - 1)
                oh[c - 1] = fire_out(c - 1)
        gh[_NCH - 1].wait()
        ph2(_NCH - 1)
        oh[_NCH - 1] = fire_out(_NCH - 1)
        for c in (_NCH - 2, _NCH - 1):
            for h in oh[c]:
                h.wait()

    src, tgt, msk = _run(node_ids, u, indptr, indices)
    return (src, tgt, msk.astype(bool))
